# baseline (speedup 1.0000x reference)
import jax
import jax.numpy as jnp
from jax.experimental import pallas as pl
from jax.experimental.pallas import tpu as pltpu

B, L, DIM = 4096, 200, 64
N = B * L
PD = 2 * DIM
NP = N // 2
BLK = 4096


def _body(b0_ref, b1_ref, r0_ref, r1_ref, pemb_ref, o_ref):
    i0 = (1 + b0_ref[...]) * (1 + ((r0_ref[...] * 3) >> 8))
    i1 = (1 + b1_ref[...]) * (1 + ((r1_ref[...] * 3) >> 8))
    pidx = i0 * 5 + i1
    onehot = (pidx[:, None] == jax.lax.broadcasted_iota(
        jnp.int32, (BLK, 32), 1)).astype(jnp.float32)
    o_ref[...] = jnp.dot(onehot, pemb_ref[...],
                         preferred_element_type=jnp.float32)


def kernel(x, emb):
    xi = x.astype(jnp.int32)
    brick = xi[..., 0].reshape(NP, 2)
    rot = xi[..., 1].reshape(NP, 2)
    pemb = jnp.zeros((32, PD), jnp.float32).at[:25].set(
        jnp.concatenate([
            jnp.broadcast_to(emb[:, None, :], (5, 5, DIM)),
            jnp.broadcast_to(emb[None, :, :], (5, 5, DIM)),
        ], axis=-1).reshape(25, PD))
    out = pl.pallas_call(
        _body,
        grid=(NP // BLK,),
        in_specs=[
            pl.BlockSpec((BLK,), lambda i: (i,)),
            pl.BlockSpec((BLK,), lambda i: (i,)),
            pl.BlockSpec((BLK,), lambda i: (i,)),
            pl.BlockSpec((BLK,), lambda i: (i,)),
            pl.BlockSpec((32, PD), lambda i: (0, 0)),
        ],
        out_specs=pl.BlockSpec((BLK, PD), lambda i: (i, 0)),
        out_shape=jax.ShapeDtypeStruct((NP, PD), jnp.float32),
    )(brick[:, 0], brick[:, 1], rot[:, 0], rot[:, 1], pemb)
    return out.reshape(B, L, DIM)
